# baseline (speedup 1.0000x reference)
import functools

import jax
import jax.numpy as jnp
from jax.experimental import pallas as pl
from jax.experimental.pallas import tpu as pltpu


_NEG = -1.0
_BIG = 2**30


def _pp_kernel(logits_ref, boxes_ref, scale_ref,
               out_s_ref, out_l_ref, out_b_ref, x_ref):
    p = jax.nn.sigmoid(logits_ref[0])
    row_iota = jax.lax.broadcasted_iota(jnp.int32, (1024, 91), 0)
    p = jnp.where(row_iota < 900, p, _NEG)
    x_ref[:, :] = p

    rowmax = jnp.max(p.reshape(8, 128, 91), axis=2)

    sub_i = jax.lax.broadcasted_iota(jnp.int32, (8, 128), 0)
    lane8_i = jax.lax.broadcasted_iota(jnp.int32, (8, 128), 1)
    flat_i = sub_i * 128 + lane8_i
    lane_i = jax.lax.broadcasted_iota(jnp.int32, (1, 91), 1)
    scale = scale_ref[0]
    lane4_i = jax.lax.broadcasted_iota(jnp.int32, (1, 4), 1)

    def body(k, rowmax):
        m = jnp.max(rowmax)
        r = jnp.min(jnp.where(rowmax == m, flat_i, _BIG))
        row = x_ref[pl.ds(r, 1), :]
        c = jnp.min(jnp.where(row == m, lane_i, _BIG))

        out_s_ref[0, pl.ds(k, 1), :] = jnp.reshape(m, (1, 1))
        out_l_ref[0, pl.ds(k, 1), :] = jnp.reshape(c, (1, 1))

        b = boxes_ref[0, pl.ds(r, 1), :]
        rolled = jnp.roll(b, 2, axis=1)
        xyxy = jnp.where(lane4_i < 2, b - 0.5 * rolled, rolled + 0.5 * b)
        out_b_ref[0, pl.ds(k, 1), :] = xyxy * scale

        new_row = jnp.where(lane_i == c, _NEG, row)
        x_ref[pl.ds(r, 1), :] = new_row
        rowmax = jnp.where(flat_i == r, jnp.max(new_row), rowmax)
        return rowmax

    jax.lax.fori_loop(0, 100, body, rowmax)


@functools.partial(jax.jit, static_argnames=("interpret",))
def kernel(pred_logits, pred_boxes, target_sizes, interpret=False):
    B, Q, C = pred_logits.shape
    logits_p = jnp.pad(pred_logits, ((0, 0), (0, 1024 - Q), (0, 0)),
                       constant_values=-1e30)
    boxes_p = jnp.pad(pred_boxes, ((0, 0), (0, 1024 - Q), (0, 0)))
    img_h = target_sizes[:, 0].astype(jnp.float32)
    img_w = target_sizes[:, 1].astype(jnp.float32)
    scale_fct = jnp.stack([img_w, img_h, img_w, img_h], axis=1)
    scale_fct = scale_fct[:, None, :]

    out_shapes = [
        jax.ShapeDtypeStruct((B, 100, 1), jnp.float32),
        jax.ShapeDtypeStruct((B, 100, 1), jnp.int32),
        jax.ShapeDtypeStruct((B, 100, 4), jnp.float32),
    ]
    s3, l3, b3 = pl.pallas_call(
        _pp_kernel,
        grid=(B,),
        in_specs=[
            pl.BlockSpec((1, 1024, C), lambda b: (b, 0, 0)),
            pl.BlockSpec((1, 1024, 4), lambda b: (b, 0, 0)),
            pl.BlockSpec((1, 1, 4), lambda b: (b, 0, 0)),
        ],
        out_specs=[
            pl.BlockSpec((1, 100, 1), lambda b: (b, 0, 0)),
            pl.BlockSpec((1, 100, 1), lambda b: (b, 0, 0)),
            pl.BlockSpec((1, 100, 4), lambda b: (b, 0, 0)),
        ],
        out_shape=out_shapes,
        scratch_shapes=[pltpu.VMEM((1024, C), jnp.float32)],
        interpret=interpret,
    )(logits_p, boxes_p, scale_fct)

    return s3[:, :, 0], l3[:, :, 0], b3


# speedup vs baseline: 1.9545x; 1.9545x over previous
import functools

import jax
import jax.numpy as jnp
from jax.experimental import pallas as pl
from jax.experimental.pallas import tpu as pltpu

_BC = 8
_QP = 904
_BIG = 2**30


def _pp_kernel(logits_ref, boxes_ref, scale_ref,
               out_s_ref, out_l_ref, out_b_ref,
               oh_ref, g_ref, c_ref):
    p = jax.nn.sigmoid(logits_ref[:, :, :])
    rowi = jax.lax.broadcasted_iota(jnp.int32, (_BC, _QP, 91), 1)
    p = jnp.where(rowi < 900, p, -1.0)

    rm = jnp.max(p, axis=2, keepdims=True)
    i904s = jax.lax.broadcasted_iota(jnp.int32, (_BC, _QP, 1), 1)
    i904l = jax.lax.broadcasted_iota(jnp.int32, (_BC, 1, _QP), 2)
    col91 = jax.lax.broadcasted_iota(jnp.int32, (_BC, 1, 91), 2)

    def select_rows(k, rm):
        bm = jnp.max(rm, axis=1, keepdims=True)
        fid = jnp.min(jnp.where(rm == bm, i904s, _BIG),
                      axis=1, keepdims=True)
        oh_ref[:, pl.ds(k, 1), :] = (i904l == fid).astype(jnp.float32)
        g_ref[:, pl.ds(k, 1), :] = fid * 91 + col91
        return jnp.where(i904s == fid, -1.0, rm)

    jax.lax.fori_loop(0, 100, select_rows, rm)

    c = jax.lax.dot_general(
        oh_ref[:, :, :], p,
        dimension_numbers=(((2,), (1,)), ((0,), (0,))),
        precision=jax.lax.Precision.HIGHEST,
        preferred_element_type=jnp.float32)
    c_ref[:, :, :] = c

    lane_k = jax.lax.broadcasted_iota(jnp.int32, (_BC, 1, 128), 2)

    def peel(k, carry):
        v100, l100, q100 = carry
        cv = c_ref[:, :, :]
        gv = g_ref[:, :, :]
        bm = jnp.max(jnp.max(cv, axis=2, keepdims=True),
                     axis=1, keepdims=True)
        masked = jnp.where(cv == bm, gv, _BIG)
        fid = jnp.min(jnp.min(masked, axis=2, keepdims=True),
                      axis=1, keepdims=True)
        q3 = fid // 91
        l3 = fid - q3 * 91
        c_ref[:, :, :] = jnp.where(gv == fid, -1.0, cv)
        oh_ref[:, pl.ds(k, 1), :] = (i904l == q3).astype(jnp.float32)
        sel = lane_k == k
        v100 = jnp.where(sel, bm, v100)
        l100 = jnp.where(sel, l3, l100)
        q100 = jnp.where(sel, q3, q100)
        return v100, l100, q100

    v100 = jnp.zeros((_BC, 1, 128), jnp.float32)
    l100 = jnp.zeros((_BC, 1, 128), jnp.int32)
    q100 = jnp.zeros((_BC, 1, 128), jnp.int32)
    v100, l100, q100 = jax.lax.fori_loop(0, 100, peel, (v100, l100, q100))

    out_s_ref[:, :, :] = v100[:, :, :100]
    out_l_ref[:, :, :] = l100[:, :, :100]

    bg = jax.lax.dot_general(
        oh_ref[:, :, :], boxes_ref[:, :, :],
        dimension_numbers=(((2,), (1,)), ((0,), (0,))),
        precision=jax.lax.Precision.HIGHEST,
        preferred_element_type=jnp.float32)
    rolled = jnp.roll(bg, 2, axis=2)
    lane4 = jax.lax.broadcasted_iota(jnp.int32, (_BC, 100, 4), 2)
    xyxy = jnp.where(lane4 < 2, bg - 0.5 * rolled, rolled + 0.5 * bg)
    out_b_ref[:, :, :] = xyxy * scale_ref[:, :, :]


@functools.partial(jax.jit, static_argnames=("interpret",))
def kernel(pred_logits, pred_boxes, target_sizes, interpret=False):
    B, Q, C = pred_logits.shape
    logits_p = jnp.pad(pred_logits, ((0, 0), (0, _QP - Q), (0, 0)),
                       constant_values=-1e30)
    boxes_p = jnp.pad(pred_boxes, ((0, 0), (0, _QP - Q), (0, 0)))
    img_h = target_sizes[:, 0].astype(jnp.float32)
    img_w = target_sizes[:, 1].astype(jnp.float32)
    scale_fct = jnp.stack([img_w, img_h, img_w, img_h], axis=1)[:, None, :]

    out_shapes = [
        jax.ShapeDtypeStruct((B, 1, 100), jnp.float32),
        jax.ShapeDtypeStruct((B, 1, 100), jnp.int32),
        jax.ShapeDtypeStruct((B, 100, 4), jnp.float32),
    ]
    s3, l3, boxes = pl.pallas_call(
        _pp_kernel,
        grid=(B // _BC,),
        in_specs=[
            pl.BlockSpec((_BC, _QP, C), lambda b: (b, 0, 0)),
            pl.BlockSpec((_BC, _QP, 4), lambda b: (b, 0, 0)),
            pl.BlockSpec((_BC, 1, 4), lambda b: (b, 0, 0)),
        ],
        out_specs=[
            pl.BlockSpec((_BC, 1, 100), lambda b: (b, 0, 0)),
            pl.BlockSpec((_BC, 1, 100), lambda b: (b, 0, 0)),
            pl.BlockSpec((_BC, 100, 4), lambda b: (b, 0, 0)),
        ],
        out_shape=out_shapes,
        scratch_shapes=[
            pltpu.VMEM((_BC, 100, _QP), jnp.float32),
            pltpu.VMEM((_BC, 100, 91), jnp.int32),
            pltpu.VMEM((_BC, 100, 91), jnp.float32),
        ],
        interpret=interpret,
    )(logits_p, boxes_p, scale_fct)

    return s3[:, 0, :], l3[:, 0, :], boxes


# speedup vs baseline: 6.4776x; 3.3142x over previous
import functools

import jax
import jax.numpy as jnp
from jax.experimental import pallas as pl
from jax.experimental.pallas import tpu as pltpu

_BC = 16
_QP = 904
_BIG = 2**30


def _pp_kernel(logits_ref, boxes_ref, scale_ref,
               out_s_ref, out_l_ref, out_b_ref,
               oh_ref, g_ref, c_ref):
    p = jax.nn.sigmoid(logits_ref[:, :, :])
    qlane3 = jax.lax.broadcasted_iota(jnp.int32, (_BC, 91, _QP), 2)
    p = jnp.where(qlane3 < 900, p, -1.0)

    rm = jnp.max(p, axis=1, keepdims=True)
    i904 = jax.lax.broadcasted_iota(jnp.int32, (_BC, 1, _QP), 2)
    lane_k = jax.lax.broadcasted_iota(jnp.int32, (_BC, 1, 128), 2)

    def select_rows(k, carry):
        rm, qsel = carry
        bm = jnp.max(rm, axis=2, keepdims=True)
        fid = jnp.min(jnp.where(rm == bm, i904, _BIG),
                      axis=2, keepdims=True)
        oh_ref[:, pl.ds(k, 1), :] = (i904 == fid).astype(jnp.float32)
        qsel = jnp.where(lane_k == k, fid, qsel)
        return jnp.where(i904 == fid, -1.0, rm), qsel

    qsel0 = jnp.zeros((_BC, 1, 128), jnp.int32)
    _, qsel = jax.lax.fori_loop(0, 100, select_rows, (rm, qsel0))

    cls91 = jax.lax.broadcasted_iota(jnp.int32, (_BC, 91, 1), 1)
    g_ref[:, :, :] = qsel[:, :, :100] * 91 + cls91

    c_ref[:, :, :] = jax.lax.dot_general(
        p, oh_ref[:, :, :],
        dimension_numbers=(((2,), (2,)), ((0,), (0,))),
        precision=jax.lax.Precision.HIGHEST,
        preferred_element_type=jnp.float32)

    def peel(k, carry):
        v100, l100, q100 = carry
        cv = c_ref[:, :, :]
        gv = g_ref[:, :, :]
        bm = jnp.max(jnp.max(cv, axis=2, keepdims=True),
                     axis=1, keepdims=True)
        masked = jnp.where(cv == bm, gv, _BIG)
        fid = jnp.min(jnp.min(masked, axis=2, keepdims=True),
                      axis=1, keepdims=True)
        q3 = fid // 91
        l3 = fid - q3 * 91
        c_ref[:, :, :] = jnp.where(gv == fid, -1.0, cv)
        oh_ref[:, pl.ds(k, 1), :] = (i904 == q3).astype(jnp.float32)
        sel = lane_k == k
        v100 = jnp.where(sel, bm, v100)
        l100 = jnp.where(sel, l3, l100)
        q100 = jnp.where(sel, q3, q100)
        return v100, l100, q100

    v100 = jnp.zeros((_BC, 1, 128), jnp.float32)
    l100 = jnp.zeros((_BC, 1, 128), jnp.int32)
    q100 = jnp.zeros((_BC, 1, 128), jnp.int32)
    v100, l100, q100 = jax.lax.fori_loop(0, 100, peel, (v100, l100, q100))

    out_s_ref[:, :, :] = v100[:, :, :100]
    out_l_ref[:, :, :] = l100[:, :, :100]

    bg = jax.lax.dot_general(
        boxes_ref[:, :, :], oh_ref[:, :, :],
        dimension_numbers=(((2,), (2,)), ((0,), (0,))),
        precision=jax.lax.Precision.HIGHEST,
        preferred_element_type=jnp.float32)
    rolled = jnp.roll(bg, 2, axis=1)
    sub4 = jax.lax.broadcasted_iota(jnp.int32, (_BC, 4, 100), 1)
    xyxy = jnp.where(sub4 < 2, bg - 0.5 * rolled, rolled + 0.5 * bg)
    out_b_ref[:, :, :] = xyxy * scale_ref[:, :, :]


@functools.partial(jax.jit, static_argnames=("interpret",))
def kernel(pred_logits, pred_boxes, target_sizes, interpret=False):
    B, Q, C = pred_logits.shape
    logits_t = jnp.pad(jnp.transpose(pred_logits, (0, 2, 1)),
                       ((0, 0), (0, 0), (0, _QP - Q)),
                       constant_values=-1e30)
    boxes_t = jnp.pad(jnp.transpose(pred_boxes, (0, 2, 1)),
                      ((0, 0), (0, 0), (0, _QP - Q)))
    img_h = target_sizes[:, 0].astype(jnp.float32)
    img_w = target_sizes[:, 1].astype(jnp.float32)
    scale_t = jnp.stack([img_w, img_h, img_w, img_h], axis=1)[:, :, None]

    out_shapes = [
        jax.ShapeDtypeStruct((B, 1, 100), jnp.float32),
        jax.ShapeDtypeStruct((B, 1, 100), jnp.int32),
        jax.ShapeDtypeStruct((B, 4, 100), jnp.float32),
    ]
    s3, l3, b3 = pl.pallas_call(
        _pp_kernel,
        grid=(B // _BC,),
        in_specs=[
            pl.BlockSpec((_BC, C, _QP), lambda b: (b, 0, 0)),
            pl.BlockSpec((_BC, 4, _QP), lambda b: (b, 0, 0)),
            pl.BlockSpec((_BC, 4, 1), lambda b: (b, 0, 0)),
        ],
        out_specs=[
            pl.BlockSpec((_BC, 1, 100), lambda b: (b, 0, 0)),
            pl.BlockSpec((_BC, 1, 100), lambda b: (b, 0, 0)),
            pl.BlockSpec((_BC, 4, 100), lambda b: (b, 0, 0)),
        ],
        out_shape=out_shapes,
        scratch_shapes=[
            pltpu.VMEM((_BC, 100, _QP), jnp.float32),
            pltpu.VMEM((_BC, 91, 100), jnp.int32),
            pltpu.VMEM((_BC, 91, 100), jnp.float32),
        ],
        interpret=interpret,
    )(logits_t, boxes_t, scale_t)

    return s3[:, 0, :], l3[:, 0, :], jnp.transpose(b3, (0, 2, 1))


# speedup vs baseline: 7.3590x; 1.1361x over previous
import functools

import jax
import jax.numpy as jnp
from jax.experimental import pallas as pl
from jax.experimental.pallas import tpu as pltpu

_BC = 32
_QP = 904
_BIG = 2**30


def _pp_kernel(logits_ref, boxes_ref, scale_ref,
               out_s_ref, out_l_ref, out_b_ref,
               oh_ref, g_ref, c_ref):
    p = jax.nn.sigmoid(logits_ref[:, :, :])
    qlane3 = jax.lax.broadcasted_iota(jnp.int32, (_BC, 91, _QP), 2)
    p = jnp.where(qlane3 < 900, p, -1.0)

    rm = jnp.max(p, axis=1, keepdims=True)
    i904 = jax.lax.broadcasted_iota(jnp.int32, (_BC, 1, _QP), 2)
    lane_k = jax.lax.broadcasted_iota(jnp.int32, (_BC, 1, 128), 2)

    def select_rows(k, carry):
        rm, qsel = carry
        bm = jnp.max(rm, axis=2, keepdims=True)
        fid = jnp.min(jnp.where(rm == bm, i904, _BIG),
                      axis=2, keepdims=True)
        oh_ref[:, pl.ds(k, 1), :] = (i904 == fid).astype(jnp.float32)
        qsel = jnp.where(lane_k == k, fid, qsel)
        return jnp.where(i904 == fid, -1.0, rm), qsel

    qsel0 = jnp.zeros((_BC, 1, 128), jnp.int32)
    _, qsel = jax.lax.fori_loop(0, 100, select_rows, (rm, qsel0))

    cls91 = jax.lax.broadcasted_iota(jnp.int32, (_BC, 91, 1), 1)
    g_ref[:, :, :] = qsel[:, :, :100] * 91 + cls91

    c_ref[:, :, :] = jax.lax.dot_general(
        p, oh_ref[:, :, :],
        dimension_numbers=(((2,), (2,)), ((0,), (0,))),
        precision=jax.lax.Precision.HIGHEST,
        preferred_element_type=jnp.float32)

    def peel(k, carry):
        v100, l100, q100 = carry
        cv = c_ref[:, :, :]
        gv = g_ref[:, :, :]
        bm = jnp.max(jnp.max(cv, axis=2, keepdims=True),
                     axis=1, keepdims=True)
        masked = jnp.where(cv == bm, gv, _BIG)
        fid = jnp.min(jnp.min(masked, axis=2, keepdims=True),
                      axis=1, keepdims=True)
        q3 = fid // 91
        l3 = fid - q3 * 91
        c_ref[:, :, :] = jnp.where(gv == fid, -1.0, cv)
        oh_ref[:, pl.ds(k, 1), :] = (i904 == q3).astype(jnp.float32)
        sel = lane_k == k
        v100 = jnp.where(sel, bm, v100)
        l100 = jnp.where(sel, l3, l100)
        q100 = jnp.where(sel, q3, q100)
        return v100, l100, q100

    v100 = jnp.zeros((_BC, 1, 128), jnp.float32)
    l100 = jnp.zeros((_BC, 1, 128), jnp.int32)
    q100 = jnp.zeros((_BC, 1, 128), jnp.int32)
    v100, l100, q100 = jax.lax.fori_loop(0, 100, peel, (v100, l100, q100))

    out_s_ref[:, :, :] = v100[:, :, :100]
    out_l_ref[:, :, :] = l100[:, :, :100]

    bg = jax.lax.dot_general(
        boxes_ref[:, :, :], oh_ref[:, :, :],
        dimension_numbers=(((2,), (2,)), ((0,), (0,))),
        precision=jax.lax.Precision.HIGHEST,
        preferred_element_type=jnp.float32)
    rolled = jnp.roll(bg, 2, axis=1)
    sub4 = jax.lax.broadcasted_iota(jnp.int32, (_BC, 4, 100), 1)
    xyxy = jnp.where(sub4 < 2, bg - 0.5 * rolled, rolled + 0.5 * bg)
    out_b_ref[:, :, :] = xyxy * scale_ref[:, :, :]


@functools.partial(jax.jit, static_argnames=("interpret",))
def kernel(pred_logits, pred_boxes, target_sizes, interpret=False):
    B, Q, C = pred_logits.shape
    logits_t = jnp.pad(jnp.transpose(pred_logits, (0, 2, 1)),
                       ((0, 0), (0, 0), (0, _QP - Q)),
                       constant_values=-1e30)
    boxes_t = jnp.pad(jnp.transpose(pred_boxes, (0, 2, 1)),
                      ((0, 0), (0, 0), (0, _QP - Q)))
    img_h = target_sizes[:, 0].astype(jnp.float32)
    img_w = target_sizes[:, 1].astype(jnp.float32)
    scale_t = jnp.stack([img_w, img_h, img_w, img_h], axis=1)[:, :, None]

    out_shapes = [
        jax.ShapeDtypeStruct((B, 1, 100), jnp.float32),
        jax.ShapeDtypeStruct((B, 1, 100), jnp.int32),
        jax.ShapeDtypeStruct((B, 4, 100), jnp.float32),
    ]
    s3, l3, b3 = pl.pallas_call(
        _pp_kernel,
        grid=(B // _BC,),
        in_specs=[
            pl.BlockSpec((_BC, C, _QP), lambda b: (b, 0, 0)),
            pl.BlockSpec((_BC, 4, _QP), lambda b: (b, 0, 0)),
            pl.BlockSpec((_BC, 4, 1), lambda b: (b, 0, 0)),
        ],
        out_specs=[
            pl.BlockSpec((_BC, 1, 100), lambda b: (b, 0, 0)),
            pl.BlockSpec((_BC, 1, 100), lambda b: (b, 0, 0)),
            pl.BlockSpec((_BC, 4, 100), lambda b: (b, 0, 0)),
        ],
        out_shape=out_shapes,
        scratch_shapes=[
            pltpu.VMEM((_BC, 100, _QP), jnp.float32),
            pltpu.VMEM((_BC, 91, 100), jnp.int32),
            pltpu.VMEM((_BC, 91, 100), jnp.float32),
        ],
        interpret=interpret,
    )(logits_t, boxes_t, scale_t)

    return s3[:, 0, :], l3[:, 0, :], jnp.transpose(b3, (0, 2, 1))


# speedup vs baseline: 7.8388x; 1.0652x over previous
import functools

import jax
import jax.numpy as jnp
from jax.experimental import pallas as pl
from jax.experimental.pallas import tpu as pltpu

_BC = 32
_QP = 904
_BIG = 2**30


def _pp_kernel(logits_ref, boxes_ref, scale_ref,
               out_s_ref, out_l_ref, out_b_ref,
               oh_ref, g_ref, c_ref):
    p = jax.nn.sigmoid(logits_ref[:, :, :])
    qlane3 = jax.lax.broadcasted_iota(jnp.int32, (_BC, 91, _QP), 2)
    p = jnp.where(qlane3 < 900, p, -1.0)

    rm = jnp.reshape(jnp.max(p, axis=1, keepdims=True), (_BC, 8, 113))
    qpack = (jax.lax.broadcasted_iota(jnp.int32, (_BC, 8, 113), 1) * 113
             + jax.lax.broadcasted_iota(jnp.int32, (_BC, 8, 113), 2))
    i904 = jax.lax.broadcasted_iota(jnp.int32, (_BC, 1, _QP), 2)
    lane_k = jax.lax.broadcasted_iota(jnp.int32, (_BC, 1, 128), 2)

    def select_rows(k, carry):
        rm, qsel = carry
        bm = jnp.max(jnp.max(rm, axis=2, keepdims=True),
                     axis=1, keepdims=True)
        fid = jnp.min(jnp.min(jnp.where(rm == bm, qpack, _BIG),
                              axis=2, keepdims=True),
                      axis=1, keepdims=True)
        oh_ref[:, pl.ds(k, 1), :] = (i904 == fid).astype(jnp.float32)
        qsel = jnp.where(lane_k == k, fid, qsel)
        return jnp.where(qpack == fid, -1.0, rm), qsel

    qsel0 = jnp.zeros((_BC, 1, 128), jnp.int32)
    _, qsel = jax.lax.fori_loop(0, 100, select_rows, (rm, qsel0))

    cls91 = jax.lax.broadcasted_iota(jnp.int32, (_BC, 91, 1), 1)
    g_ref[:, :, :] = qsel[:, :, :100] * 91 + cls91

    c_ref[:, :, :] = jax.lax.dot_general(
        p, oh_ref[:, :, :],
        dimension_numbers=(((2,), (2,)), ((0,), (0,))),
        precision=jax.lax.Precision.HIGHEST,
        preferred_element_type=jnp.float32)

    def peel(k, carry):
        v100, l100, q100 = carry
        cv = c_ref[:, :, :]
        gv = g_ref[:, :, :]
        bm = jnp.max(jnp.max(cv, axis=2, keepdims=True),
                     axis=1, keepdims=True)
        masked = jnp.where(cv == bm, gv, _BIG)
        fid = jnp.min(jnp.min(masked, axis=2, keepdims=True),
                      axis=1, keepdims=True)
        q3 = fid // 91
        l3 = fid - q3 * 91
        c_ref[:, :, :] = jnp.where(gv == fid, -1.0, cv)
        oh_ref[:, pl.ds(k, 1), :] = (i904 == q3).astype(jnp.float32)
        sel = lane_k == k
        v100 = jnp.where(sel, bm, v100)
        l100 = jnp.where(sel, l3, l100)
        q100 = jnp.where(sel, q3, q100)
        return v100, l100, q100

    v100 = jnp.zeros((_BC, 1, 128), jnp.float32)
    l100 = jnp.zeros((_BC, 1, 128), jnp.int32)
    q100 = jnp.zeros((_BC, 1, 128), jnp.int32)
    v100, l100, q100 = jax.lax.fori_loop(0, 100, peel, (v100, l100, q100))

    out_s_ref[:, :, :] = v100[:, :, :100]
    out_l_ref[:, :, :] = l100[:, :, :100]

    bg = jax.lax.dot_general(
        boxes_ref[:, :, :], oh_ref[:, :, :],
        dimension_numbers=(((2,), (2,)), ((0,), (0,))),
        precision=jax.lax.Precision.HIGHEST,
        preferred_element_type=jnp.float32)
    rolled = jnp.roll(bg, 2, axis=1)
    sub4 = jax.lax.broadcasted_iota(jnp.int32, (_BC, 4, 100), 1)
    xyxy = jnp.where(sub4 < 2, bg - 0.5 * rolled, rolled + 0.5 * bg)
    out_b_ref[:, :, :] = xyxy * scale_ref[:, :, :]


@functools.partial(jax.jit, static_argnames=("interpret",))
def kernel(pred_logits, pred_boxes, target_sizes, interpret=False):
    B, Q, C = pred_logits.shape
    logits_t = jnp.pad(jnp.transpose(pred_logits, (0, 2, 1)),
                       ((0, 0), (0, 0), (0, _QP - Q)),
                       constant_values=-1e30)
    boxes_t = jnp.pad(jnp.transpose(pred_boxes, (0, 2, 1)),
                      ((0, 0), (0, 0), (0, _QP - Q)))
    img_h = target_sizes[:, 0].astype(jnp.float32)
    img_w = target_sizes[:, 1].astype(jnp.float32)
    scale_t = jnp.stack([img_w, img_h, img_w, img_h], axis=1)[:, :, None]

    out_shapes = [
        jax.ShapeDtypeStruct((B, 1, 100), jnp.float32),
        jax.ShapeDtypeStruct((B, 1, 100), jnp.int32),
        jax.ShapeDtypeStruct((B, 4, 100), jnp.float32),
    ]
    s3, l3, b3 = pl.pallas_call(
        _pp_kernel,
        grid=(B // _BC,),
        in_specs=[
            pl.BlockSpec((_BC, C, _QP), lambda b: (b, 0, 0)),
            pl.BlockSpec((_BC, 4, _QP), lambda b: (b, 0, 0)),
            pl.BlockSpec((_BC, 4, 1), lambda b: (b, 0, 0)),
        ],
        out_specs=[
            pl.BlockSpec((_BC, 1, 100), lambda b: (b, 0, 0)),
            pl.BlockSpec((_BC, 1, 100), lambda b: (b, 0, 0)),
            pl.BlockSpec((_BC, 4, 100), lambda b: (b, 0, 0)),
        ],
        out_shape=out_shapes,
        scratch_shapes=[
            pltpu.VMEM((_BC, 100, _QP), jnp.float32),
            pltpu.VMEM((_BC, 91, 100), jnp.int32),
            pltpu.VMEM((_BC, 91, 100), jnp.float32),
        ],
        interpret=interpret,
    )(logits_t, boxes_t, scale_t)

    return s3[:, 0, :], l3[:, 0, :], jnp.transpose(b3, (0, 2, 1))
